# baseline (speedup 1.0000x reference)
import functools
import jax
import jax.numpy as jnp
from jax import lax
from jax.experimental import pallas as pl
from jax.experimental.pallas import tpu as pltpu
from jax.experimental.pallas import tpu_sc as plsc

T = 15
C_IN = 64
C_OUT = 128
H_IN = 96
W_IN = 96
KH = 5
KW = 5
H_OUT = H_IN - KH + 1
W_OUT = W_IN - KW + 1
N_WIN = 16
LOWER = 0.0
UPPER = 1.0

CS = 16
CSC = C_IN - CS
QH = 24
NQ = H_IN // QH
UNITS_PER_TILE = (CSC * NQ) // 32


def _sc_reduce(xs, out, buf_a, buf_b, acc_a, acc_b, sem_a, sem_b):
    cid = lax.axis_index("c")
    sid = lax.axis_index("s")
    wid = sid * 2 + cid

    bufs = (buf_a, buf_b)
    accs = (acc_a, acc_b)
    sems = (sem_a, sem_b)

    def unit_cq(u):
        g = wid * UNITS_PER_TILE + u
        return CS + g // NQ, (g % NQ) * QH

    def fire(u, b):
        c, q = unit_cq(u)
        for t in range(T):
            pltpu.make_async_copy(
                xs.at[t, c, pl.ds(q, QH), :], bufs[b].at[t], sems[b]
            ).start()

    def drain_compute_write(u, b):
        c, q = unit_cq(u)
        for t in range(T):
            pltpu.make_async_copy(
                xs.at[t, c, pl.ds(q, QH), :], bufs[b].at[t], sems[b]
            ).wait()
        buf = bufs[b]
        acc = accs[b]

        def body(r, _):
            for k in range(W_IN // 16):
                s = pl.ds(k * 16, 16)
                v = buf[0, r, s]
                for t in range(1, T):
                    v = v + buf[t, r, s]
                acc[r, s] = v
            return _

        lax.fori_loop(0, QH, body, None)
        pltpu.make_async_copy(
            acc, out.at[c - CS, pl.ds(q, QH), :], sems[b]
        ).start()
        pltpu.make_async_copy(
            acc, out.at[c - CS, pl.ds(q, QH), :], sems[b]
        ).wait()

    fire(0, 0)
    for u in range(UNITS_PER_TILE):
        if u + 1 < UNITS_PER_TILE:
            fire(u + 1, (u + 1) % 2)
        drain_compute_write(u, u % 2)


def _tc_reduce(x_ref, o_ref):
    t = pl.program_id(0)

    @pl.when(t == 0)
    def _():
        o_ref[...] = x_ref[0]

    @pl.when(t > 0)
    def _():
        o_ref[...] += x_ref[0]


def _tail_kernel(win_ref, ltc_ref, lsc_ref, os_ref, w_ref, ltp_ref, ltd_ref,
                 out_ref, lat_ref, ov_ref, osems):
    for i in range(N_WIN):
        f = win_ref[i, 0]
        h = win_ref[i, 1]
        h8 = jnp.minimum((h // 8) * 8, H_OUT - 8)
        pltpu.make_async_copy(
            os_ref.at[:, pl.ds(f, 1), pl.ds(h8, 8), :],
            ov_ref.at[i], osems.at[i],
        ).start()

    for hh in range(H_IN):
        lat_ref[hh] = jnp.concatenate(
            [jnp.transpose(ltc_ref[:, hh, :], (1, 0)),
             jnp.transpose(lsc_ref[:, hh, :], (1, 0))], axis=1)

    out_ref[...] = jnp.clip(w_ref[...], LOWER, UPPER)

    sub = jax.lax.broadcasted_iota(jnp.int32, (T, 1, 8, W_OUT), 2)
    lane = jax.lax.broadcasted_iota(jnp.int32, (T, 1, 8, W_OUT), 3)
    for i in range(N_WIN):
        f = win_ref[i, 0]
        h = win_ref[i, 1]
        w = win_ref[i, 2]
        h8 = jnp.minimum((h // 8) * 8, H_OUT - 8)
        pltpu.make_async_copy(
            os_ref.at[:, pl.ds(f, 1), pl.ds(h8, 8), :],
            ov_ref.at[i], osems.at[i],
        ).wait()
        out_val = jnp.sum(
            jnp.where((sub == h - h8) & (lane == w), ov_ref[i], 0.0))
        pieces = []
        for kh in range(KH):
            pieces.append(lat_ref[h + kh, pl.ds(w, KW), :])
        patch = jnp.concatenate(pieces, axis=0)
        patch_t = jnp.transpose(patch, (1, 0))
        wv = w_ref[f]
        row = jnp.where(patch_t >= out_val, ltp_ref[f], ltd_ref[f])
        stab = (wv - LOWER) * (UPPER - wv)
        out_ref[f] = jnp.clip(wv + row * stab, LOWER, UPPER)


def kernel(input_spikes, potentials, output_spikes, winners, weight, ltp, ltd):
    del potentials
    w2 = weight.reshape(C_OUT, C_IN, KH * KW)
    mesh = plsc.VectorSubcoreMesh(core_axis_name="c", subcore_axis_name="s")

    sc_fn = functools.partial(
        pl.kernel,
        out_type=jax.ShapeDtypeStruct((CSC, H_IN, W_IN), jnp.float32),
        mesh=mesh,
        scratch_types=[
            pltpu.VMEM((T, QH, W_IN), jnp.float32),
            pltpu.VMEM((T, QH, W_IN), jnp.float32),
            pltpu.VMEM((QH, W_IN), jnp.float32),
            pltpu.VMEM((QH, W_IN), jnp.float32),
            pltpu.SemaphoreType.DMA,
            pltpu.SemaphoreType.DMA,
        ],
    )(_sc_reduce)
    lat_sc = sc_fn(input_spikes)

    lat_tc = pl.pallas_call(
        _tc_reduce,
        grid=(T,),
        in_specs=[pl.BlockSpec((1, CS, H_IN, W_IN), lambda t: (t, 0, 0, 0))],
        out_specs=pl.BlockSpec((CS, H_IN, W_IN), lambda t: (0, 0, 0)),
        out_shape=jax.ShapeDtypeStruct((CS, H_IN, W_IN), jnp.float32),
    )(input_spikes)

    out2 = pl.pallas_call(
        _tail_kernel,
        grid_spec=pltpu.PrefetchScalarGridSpec(
            num_scalar_prefetch=1,
            grid=(1,),
            in_specs=[
                pl.BlockSpec((CS, H_IN, W_IN), lambda i, win: (0, 0, 0)),
                pl.BlockSpec((CSC, H_IN, W_IN), lambda i, win: (0, 0, 0)),
                pl.BlockSpec(memory_space=pl.ANY),
                pl.BlockSpec((C_OUT, C_IN, KH * KW),
                             lambda i, win: (0, 0, 0)),
                pl.BlockSpec(memory_space=pltpu.SMEM),
                pl.BlockSpec(memory_space=pltpu.SMEM),
            ],
            out_specs=pl.BlockSpec((C_OUT, C_IN, KH * KW),
                                   lambda i, win: (0, 0, 0)),
            scratch_shapes=[
                pltpu.VMEM((H_IN, W_IN, C_IN), jnp.float32),
                pltpu.VMEM((N_WIN, T, 1, 8, W_OUT), jnp.float32),
                pltpu.SemaphoreType.DMA((N_WIN,)),
            ],
        ),
        out_shape=jax.ShapeDtypeStruct((C_OUT, C_IN, KH * KW), jnp.float32),
    )(winners, lat_tc, lat_sc, output_spikes, w2, ltp, ltd)
    return out2.reshape(C_OUT, C_IN, KH, KW)


# speedup vs baseline: 1.0680x; 1.0680x over previous
import functools
import jax
import jax.numpy as jnp
from jax import lax
from jax.experimental import pallas as pl
from jax.experimental.pallas import tpu as pltpu
from jax.experimental.pallas import tpu_sc as plsc

T = 15
C_IN = 64
C_OUT = 128
H_IN = 96
W_IN = 96
KH = 5
KW = 5
H_OUT = H_IN - KH + 1
W_OUT = W_IN - KW + 1
N_WIN = 16
LOWER = 0.0
UPPER = 1.0

CS = 0
CSC = C_IN - CS
QH = 24
NQ = H_IN // QH
UNITS_PER_TILE = (CSC * NQ) // 32


def _sc_reduce(xs, out, buf_a, buf_b, acc_a, acc_b, sem_a, sem_b):
    cid = lax.axis_index("c")
    sid = lax.axis_index("s")
    wid = sid * 2 + cid

    bufs = (buf_a, buf_b)
    accs = (acc_a, acc_b)
    sems = (sem_a, sem_b)

    def unit_cq(u):
        g = wid * UNITS_PER_TILE + u
        return CS + g // NQ, (g % NQ) * QH

    def fire(u, b):
        c, q = unit_cq(u)
        for t in range(T):
            pltpu.make_async_copy(
                xs.at[t, c, pl.ds(q, QH), :], bufs[b].at[t], sems[b]
            ).start()

    def drain_compute_write(u, b):
        c, q = unit_cq(u)
        for t in range(T):
            pltpu.make_async_copy(
                xs.at[t, c, pl.ds(q, QH), :], bufs[b].at[t], sems[b]
            ).wait()
        buf = bufs[b]
        acc = accs[b]

        def body(r, _):
            for k in range(W_IN // 16):
                s = pl.ds(k * 16, 16)
                v = buf[0, r, s]
                for t in range(1, T):
                    v = v + buf[t, r, s]
                acc[r, s] = v
            return _

        lax.fori_loop(0, QH, body, None)
        pltpu.make_async_copy(
            acc, out.at[c - CS, pl.ds(q, QH), :], sems[b]
        ).start()
        pltpu.make_async_copy(
            acc, out.at[c - CS, pl.ds(q, QH), :], sems[b]
        ).wait()

    fire(0, 0)
    for u in range(UNITS_PER_TILE):
        if u + 1 < UNITS_PER_TILE:
            fire(u + 1, (u + 1) % 2)
        drain_compute_write(u, u % 2)


def _tc_reduce(x_ref, o_ref):
    t = pl.program_id(0)

    @pl.when(t == 0)
    def _():
        o_ref[...] = x_ref[0]

    @pl.when(t > 0)
    def _():
        o_ref[...] += x_ref[0]


def _tail_kernel(win_ref, lsc_ref, os_ref, w_ref, ltp_ref, ltd_ref,
                 out_ref, lat_ref, ov_ref, osems):
    for i in range(N_WIN):
        f = win_ref[i, 0]
        h = win_ref[i, 1]
        h8 = jnp.minimum((h // 8) * 8, H_OUT - 8)
        pltpu.make_async_copy(
            os_ref.at[:, pl.ds(f, 1), pl.ds(h8, 8), :],
            ov_ref.at[i], osems.at[i],
        ).start()

    for hh in range(H_IN):
        lat_ref[hh] = jnp.transpose(lsc_ref[:, hh, :], (1, 0))

    out_ref[...] = jnp.clip(w_ref[...], LOWER, UPPER)

    sub = jax.lax.broadcasted_iota(jnp.int32, (T, 1, 8, W_OUT), 2)
    lane = jax.lax.broadcasted_iota(jnp.int32, (T, 1, 8, W_OUT), 3)
    for i in range(N_WIN):
        f = win_ref[i, 0]
        h = win_ref[i, 1]
        w = win_ref[i, 2]
        h8 = jnp.minimum((h // 8) * 8, H_OUT - 8)
        pltpu.make_async_copy(
            os_ref.at[:, pl.ds(f, 1), pl.ds(h8, 8), :],
            ov_ref.at[i], osems.at[i],
        ).wait()
        out_val = jnp.sum(
            jnp.where((sub == h - h8) & (lane == w), ov_ref[i], 0.0))
        pieces = []
        for kh in range(KH):
            pieces.append(lat_ref[h + kh, pl.ds(w, KW), :])
        patch = jnp.concatenate(pieces, axis=0)
        patch_t = jnp.transpose(patch, (1, 0))
        wv = w_ref[f]
        row = jnp.where(patch_t >= out_val, ltp_ref[f], ltd_ref[f])
        stab = (wv - LOWER) * (UPPER - wv)
        out_ref[f] = jnp.clip(wv + row * stab, LOWER, UPPER)


def kernel(input_spikes, potentials, output_spikes, winners, weight, ltp, ltd):
    del potentials
    w2 = weight.reshape(C_OUT, C_IN, KH * KW)
    mesh = plsc.VectorSubcoreMesh(core_axis_name="c", subcore_axis_name="s")

    sc_fn = functools.partial(
        pl.kernel,
        out_type=jax.ShapeDtypeStruct((CSC, H_IN, W_IN), jnp.float32),
        mesh=mesh,
        scratch_types=[
            pltpu.VMEM((T, QH, W_IN), jnp.float32),
            pltpu.VMEM((T, QH, W_IN), jnp.float32),
            pltpu.VMEM((QH, W_IN), jnp.float32),
            pltpu.VMEM((QH, W_IN), jnp.float32),
            pltpu.SemaphoreType.DMA,
            pltpu.SemaphoreType.DMA,
        ],
    )(_sc_reduce)
    lat_sc = sc_fn(input_spikes)

    out2 = pl.pallas_call(
        _tail_kernel,
        grid_spec=pltpu.PrefetchScalarGridSpec(
            num_scalar_prefetch=1,
            grid=(1,),
            in_specs=[
                pl.BlockSpec((CSC, H_IN, W_IN), lambda i, win: (0, 0, 0)),
                pl.BlockSpec(memory_space=pl.ANY),
                pl.BlockSpec((C_OUT, C_IN, KH * KW),
                             lambda i, win: (0, 0, 0)),
                pl.BlockSpec(memory_space=pltpu.SMEM),
                pl.BlockSpec(memory_space=pltpu.SMEM),
            ],
            out_specs=pl.BlockSpec((C_OUT, C_IN, KH * KW),
                                   lambda i, win: (0, 0, 0)),
            scratch_shapes=[
                pltpu.VMEM((H_IN, W_IN, C_IN), jnp.float32),
                pltpu.VMEM((N_WIN, T, 1, 8, W_OUT), jnp.float32),
                pltpu.SemaphoreType.DMA((N_WIN,)),
            ],
        ),
        out_shape=jax.ShapeDtypeStruct((C_OUT, C_IN, KH * KW), jnp.float32),
    )(winners, lat_sc, output_spikes, w2, ltp, ltd)
    return out2.reshape(C_OUT, C_IN, KH, KW)


# speedup vs baseline: 1.1167x; 1.0456x over previous
import functools
import jax
import jax.numpy as jnp
from jax import lax
from jax.experimental import pallas as pl
from jax.experimental.pallas import tpu as pltpu
from jax.experimental.pallas import tpu_sc as plsc

T = 15
C_IN = 64
C_OUT = 128
H_IN = 96
W_IN = 96
KH = 5
KW = 5
H_OUT = H_IN - KH + 1
W_OUT = W_IN - KW + 1
N_WIN = 16
LOWER = 0.0
UPPER = 1.0

CS = 0
CSC = C_IN - CS
QH = 24
NQ = H_IN // QH
UNITS_PER_TILE = (CSC * NQ) // 32


def _sc_reduce(xs, out, buf_a, buf_b, acc_a, acc_b, sem_a, sem_b):
    cid = lax.axis_index("c")
    sid = lax.axis_index("s")
    wid = sid * 2 + cid

    bufs = (buf_a, buf_b)
    accs = (acc_a, acc_b)
    sems = (sem_a, sem_b)

    def unit_cq(u):
        g = wid * UNITS_PER_TILE + u
        return CS + g // NQ, (g % NQ) * QH

    def fire(u, b):
        c, q = unit_cq(u)
        for t in range(T):
            pltpu.make_async_copy(
                xs.at[t, c, pl.ds(q, QH), :], bufs[b].at[t], sems[b]
            ).start()

    def drain_compute_write(u, b):
        c, q = unit_cq(u)
        for t in range(T):
            pltpu.make_async_copy(
                xs.at[t, c, pl.ds(q, QH), :], bufs[b].at[t], sems[b]
            ).wait()
        buf = bufs[b]
        acc = accs[b]

        def body(r, _):
            for k in range(W_IN // 16):
                s = pl.ds(k * 16, 16)
                v = buf[0, r, s]
                for t in range(1, T):
                    v = v + buf[t, r, s]
                acc[r, s] = v
            return _

        lax.fori_loop(0, QH, body, None)
        pltpu.make_async_copy(
            acc, out.at[c - CS, pl.ds(q, QH), :], sems[b]
        ).start()
        pltpu.make_async_copy(
            acc, out.at[c - CS, pl.ds(q, QH), :], sems[b]
        ).wait()

    fire(0, 0)
    for u in range(UNITS_PER_TILE):
        if u + 1 < UNITS_PER_TILE:
            fire(u + 1, (u + 1) % 2)
        drain_compute_write(u, u % 2)


def _tc_reduce(x_ref, o_ref):
    t = pl.program_id(0)

    @pl.when(t == 0)
    def _():
        o_ref[...] = x_ref[0]

    @pl.when(t > 0)
    def _():
        o_ref[...] += x_ref[0]


def _tail_kernel(win_ref, lsc_ref, os_ref, w_ref, ltp_ref, ltd_ref,
                 out_ref, lat_ref, ov_ref, osems):
    for i in range(N_WIN):
        f = win_ref[i, 0]
        h = win_ref[i, 1]
        h8 = jnp.minimum((h // 8) * 8, H_OUT - 8)
        pltpu.make_async_copy(
            os_ref.at[:, pl.ds(f, 1), pl.ds(h8, 8), :],
            ov_ref.at[i], osems.at[i],
        ).start()

    for hh in range(H_IN):
        lat_ref[hh] = jnp.transpose(lsc_ref[:, hh, :], (1, 0))

    out_ref[...] = jnp.clip(w_ref[...], LOWER, UPPER)

    sub = jax.lax.broadcasted_iota(jnp.int32, (T, 1, 8, W_OUT), 2)
    lane = jax.lax.broadcasted_iota(jnp.int32, (T, 1, 8, W_OUT), 3)
    for i in range(N_WIN):
        f = win_ref[i, 0]
        h = win_ref[i, 1]
        w = win_ref[i, 2]
        h8 = jnp.minimum((h // 8) * 8, H_OUT - 8)
        pltpu.make_async_copy(
            os_ref.at[:, pl.ds(f, 1), pl.ds(h8, 8), :],
            ov_ref.at[i], osems.at[i],
        ).wait()
        out_val = jnp.sum(
            jnp.where((sub == h - h8) & (lane == w), ov_ref[i], 0.0))
        pieces = []
        for kh in range(KH):
            pieces.append(lat_ref[h + kh, pl.ds(w, KW), :])
        patch = jnp.concatenate(pieces, axis=0)
        wv = w_ref[f]
        row = jnp.where(patch >= out_val, ltp_ref[f], ltd_ref[f])
        stab = (wv - LOWER) * (UPPER - wv)
        out_ref[f] = jnp.clip(wv + row * stab, LOWER, UPPER)


def kernel(input_spikes, potentials, output_spikes, winners, weight, ltp, ltd):
    del potentials
    w2 = weight.transpose(0, 2, 3, 1).reshape(C_OUT, KH * KW, C_IN)
    mesh = plsc.VectorSubcoreMesh(core_axis_name="c", subcore_axis_name="s")

    sc_fn = functools.partial(
        pl.kernel,
        out_type=jax.ShapeDtypeStruct((CSC, H_IN, W_IN), jnp.float32),
        mesh=mesh,
        scratch_types=[
            pltpu.VMEM((T, QH, W_IN), jnp.float32),
            pltpu.VMEM((T, QH, W_IN), jnp.float32),
            pltpu.VMEM((QH, W_IN), jnp.float32),
            pltpu.VMEM((QH, W_IN), jnp.float32),
            pltpu.SemaphoreType.DMA,
            pltpu.SemaphoreType.DMA,
        ],
    )(_sc_reduce)
    lat_sc = sc_fn(input_spikes)

    out2 = pl.pallas_call(
        _tail_kernel,
        grid_spec=pltpu.PrefetchScalarGridSpec(
            num_scalar_prefetch=1,
            grid=(1,),
            in_specs=[
                pl.BlockSpec((CSC, H_IN, W_IN), lambda i, win: (0, 0, 0)),
                pl.BlockSpec(memory_space=pl.ANY),
                pl.BlockSpec((C_OUT, KH * KW, C_IN),
                             lambda i, win: (0, 0, 0)),
                pl.BlockSpec(memory_space=pltpu.SMEM),
                pl.BlockSpec(memory_space=pltpu.SMEM),
            ],
            out_specs=pl.BlockSpec((C_OUT, KH * KW, C_IN),
                                   lambda i, win: (0, 0, 0)),
            scratch_shapes=[
                pltpu.VMEM((H_IN, W_IN, C_IN), jnp.float32),
                pltpu.VMEM((N_WIN, T, 1, 8, W_OUT), jnp.float32),
                pltpu.SemaphoreType.DMA((N_WIN,)),
            ],
        ),
        out_shape=jax.ShapeDtypeStruct((C_OUT, KH * KW, C_IN), jnp.float32),
    )(winners, lat_sc, output_spikes, w2, ltp, ltd)
    return out2.reshape(C_OUT, KH, KW, C_IN).transpose(0, 3, 1, 2)
